# Initial kernel scaffold; baseline (speedup 1.0000x reference)
import jax
import jax.numpy as jnp
from jax.experimental import pallas as pl


def kernel(logits, noise):
    raise NotImplementedError("write your pallas kernel here")


# speedup vs baseline: 1.7768x; 1.7768x over previous
import functools

import jax
import jax.numpy as jnp
from jax.experimental import pallas as pl
from jax.experimental.pallas import tpu as pltpu

_TEMPERATURE = 1.0
_EPS = 1e-10


def _body(n_cols, block, nblocks,
          logits_ref, noise_ref,
          act_ref, logp_ref, logz_ref,
          mx_ref, arg_ref, val_ref, m_ref, s_ref):
    j = pl.program_id(0)

    @pl.when(j == 0)
    def _init():
        neg = jnp.full(mx_ref.shape, -jnp.inf, jnp.float32)
        mx_ref[...] = neg
        m_ref[...] = neg
        s_ref[...] = jnp.zeros(s_ref.shape, jnp.float32)
        arg_ref[...] = jnp.zeros(arg_ref.shape, jnp.int32)
        val_ref[...] = jnp.zeros(val_ref.shape, jnp.float32)

    l = logits_ref[...] / _TEMPERATURE
    u = noise_ref[...]
    gumbel = -jnp.log(-jnp.log(u + _EPS) + _EPS)
    pert = l + gumbel

    cols = jax.lax.broadcasted_iota(jnp.int32, l.shape, 1) + j * block
    valid = cols < n_cols
    neg_inf = jnp.float32(-jnp.inf)
    pert = jnp.where(valid, pert, neg_inf)
    lm = jnp.where(valid, l, neg_inf)

    bm = jnp.max(pert, axis=1, keepdims=True)
    is_max = pert == bm
    bidx = jnp.min(jnp.where(is_max, cols, jnp.int32(2**31 - 1)),
                   axis=1, keepdims=True)
    bval = jnp.max(jnp.where(cols == bidx, lm, neg_inf),
                   axis=1, keepdims=True)

    upd = bm > mx_ref[...]
    arg_ref[...] = jnp.where(upd, bidx, arg_ref[...])
    val_ref[...] = jnp.where(upd, bval, val_ref[...])
    mx_ref[...] = jnp.maximum(mx_ref[...], bm)

    bmax = jnp.max(lm, axis=1, keepdims=True)
    new_m = jnp.maximum(m_ref[...], bmax)
    se = jnp.sum(jnp.exp(lm - new_m), axis=1, keepdims=True)
    s_ref[...] = s_ref[...] * jnp.exp(m_ref[...] - new_m) + se
    m_ref[...] = new_m

    @pl.when(j == nblocks - 1)
    def _fin():
        logz = m_ref[...] + jnp.log(s_ref[...])
        logz_ref[...] = logz
        act_ref[...] = arg_ref[...]
        logp_ref[...] = val_ref[...] - logz


def kernel(logits, noise):
    n_rows, n_cols = logits.shape
    block = 16384
    nblocks = pl.cdiv(n_cols, block)

    out_shape = [
        jax.ShapeDtypeStruct((n_rows, 1), jnp.int32),
        jax.ShapeDtypeStruct((n_rows, 1), jnp.float32),
        jax.ShapeDtypeStruct((n_rows, 1), jnp.float32),
    ]
    acc = lambda dt: pltpu.VMEM((n_rows, 1), dt)
    actions, logp, logz = pl.pallas_call(
        functools.partial(_body, n_cols, block, nblocks),
        grid=(nblocks,),
        in_specs=[
            pl.BlockSpec((n_rows, block), lambda j: (0, j)),
            pl.BlockSpec((n_rows, block), lambda j: (0, j)),
        ],
        out_specs=[
            pl.BlockSpec((n_rows, 1), lambda j: (0, 0)),
            pl.BlockSpec((n_rows, 1), lambda j: (0, 0)),
            pl.BlockSpec((n_rows, 1), lambda j: (0, 0)),
        ],
        out_shape=out_shape,
        scratch_shapes=[acc(jnp.float32), acc(jnp.int32), acc(jnp.float32),
                        acc(jnp.float32), acc(jnp.float32)],
        compiler_params=pltpu.CompilerParams(
            dimension_semantics=("arbitrary",)),
    )(logits, noise)
    return actions[:, 0], logp[:, 0], logz[:, 0]


# speedup vs baseline: 1.8612x; 1.0475x over previous
import functools

import jax
import jax.numpy as jnp
from jax.experimental import pallas as pl
from jax.experimental.pallas import tpu as pltpu

_TEMPERATURE = 1.0
_EPS = 1e-10


def _body(n_cols, block, nblocks,
          logits_ref, noise_ref,
          act_ref, logp_ref, logz_ref,
          mx_ref, arg_ref, val_ref, m_ref, s_ref):
    j = pl.program_id(0)

    @pl.when(j == 0)
    def _init():
        neg = jnp.full(mx_ref.shape, -jnp.inf, jnp.float32)
        mx_ref[...] = neg
        m_ref[...] = neg
        s_ref[...] = jnp.zeros(s_ref.shape, jnp.float32)
        arg_ref[...] = jnp.zeros(arg_ref.shape, jnp.int32)
        val_ref[...] = jnp.zeros(val_ref.shape, jnp.float32)

    l = logits_ref[...] / _TEMPERATURE
    u = noise_ref[...]
    gumbel = -jnp.log(-jnp.log(u + _EPS) + _EPS)
    pert = l + gumbel

    cols = jax.lax.broadcasted_iota(jnp.int32, l.shape, 1) + j * block
    valid = cols < n_cols
    neg_inf = jnp.float32(-jnp.inf)
    pert = jnp.where(valid, pert, neg_inf)
    lm = jnp.where(valid, l, neg_inf)

    bm = jnp.max(pert, axis=1, keepdims=True)
    is_max = pert == bm
    bidx = jnp.min(jnp.where(is_max, cols, jnp.int32(2**31 - 1)),
                   axis=1, keepdims=True)
    bval = jnp.max(jnp.where(cols == bidx, lm, neg_inf),
                   axis=1, keepdims=True)

    upd = bm > mx_ref[...]
    arg_ref[...] = jnp.where(upd, bidx, arg_ref[...])
    val_ref[...] = jnp.where(upd, bval, val_ref[...])
    mx_ref[...] = jnp.maximum(mx_ref[...], bm)

    bmax = jnp.max(lm, axis=1, keepdims=True)
    new_m = jnp.maximum(m_ref[...], bmax)
    se = jnp.sum(jnp.exp(lm - new_m), axis=1, keepdims=True)
    s_ref[...] = s_ref[...] * jnp.exp(m_ref[...] - new_m) + se
    m_ref[...] = new_m

    @pl.when(j == nblocks - 1)
    def _fin():
        logz = m_ref[...] + jnp.log(s_ref[...])
        logz_ref[...] = logz
        act_ref[...] = arg_ref[...]
        logp_ref[...] = val_ref[...] - logz


def kernel(logits, noise):
    n_rows, n_cols = logits.shape
    block = 32768
    nblocks = pl.cdiv(n_cols, block)

    out_shape = [
        jax.ShapeDtypeStruct((n_rows, 1), jnp.int32),
        jax.ShapeDtypeStruct((n_rows, 1), jnp.float32),
        jax.ShapeDtypeStruct((n_rows, 1), jnp.float32),
    ]
    acc = lambda dt: pltpu.VMEM((n_rows, 1), dt)
    actions, logp, logz = pl.pallas_call(
        functools.partial(_body, n_cols, block, nblocks),
        grid=(nblocks,),
        in_specs=[
            pl.BlockSpec((n_rows, block), lambda j: (0, j)),
            pl.BlockSpec((n_rows, block), lambda j: (0, j)),
        ],
        out_specs=[
            pl.BlockSpec((n_rows, 1), lambda j: (0, 0)),
            pl.BlockSpec((n_rows, 1), lambda j: (0, 0)),
            pl.BlockSpec((n_rows, 1), lambda j: (0, 0)),
        ],
        out_shape=out_shape,
        scratch_shapes=[acc(jnp.float32), acc(jnp.int32), acc(jnp.float32),
                        acc(jnp.float32), acc(jnp.float32)],
        compiler_params=pltpu.CompilerParams(
            dimension_semantics=("arbitrary",)),
    )(logits, noise)
    return actions[:, 0], logp[:, 0], logz[:, 0]
